# baseline (speedup 1.0000x reference)
import functools

import jax
import jax.numpy as jnp
from jax import lax
from jax.experimental import pallas as pl
from jax.experimental.pallas import tpu as pltpu
from jax.experimental.pallas import tpu_sc as plsc

_K = 16
_CHUNK = 128


def _gelu(x):
    return 0.5 * x * (1.0 + lax.erf(x * 0.7071067811865476))


def _topk_body(pc_ref, pct_ref, idx_ref):
    n = pc_ref.shape[1]
    b = pl.program_id(0)
    pcb = pc_ref[0]
    pct = pct_ref[0]

    m = jnp.dot(pcb, pct, preferred_element_type=jnp.float32)
    r_row = jnp.sum(pcb * pcb, axis=1, keepdims=True)
    r_col = jnp.sum(pct * pct, axis=0, keepdims=True)
    d = r_row - 2.0 * m + r_col + 1e-5

    cols = lax.broadcasted_iota(jnp.int32, (n, n), 1)
    ams = []
    for t in range(_K + 1):
        minval = jnp.min(d, axis=1, keepdims=True)
        am = jnp.min(jnp.where(d == minval, cols, n), axis=1, keepdims=True)
        if t > 0:
            ams.append(am)
        d = jnp.where(cols == am, jnp.float32(jnp.inf), d)
    idx_ref[0] = jnp.concatenate(ams, axis=1) + b * n


def _mlp_body(g_ref, ft_ref, a1_ref, c1_ref, w2t_ref, b1_ref, b2_ref,
              out_ref):
    n = ft_ref.shape[1]
    knn = g_ref[0]
    ftb = ft_ref[0]
    c = jnp.dot(ftb, c1_ref[...], preferred_element_type=jnp.float32) + b1_ref[...]
    h1 = jnp.dot(knn, a1_ref[...], preferred_element_type=jnp.float32)
    h1 = _gelu(h1.reshape(n, _K, -1) + c[:, None, :])
    h2 = jnp.dot(h1.reshape(n * _K, -1), w2t_ref[...],
                 preferred_element_type=jnp.float32) + b2_ref[...]
    h2 = _gelu(h2)
    out_ref[0] = jnp.mean(h2.reshape(n, _K, -1), axis=1)


def _sc_gather(table, gidx, n_rows, f):
    info = plsc.get_sparse_core_info()
    nc, ns = info.num_cores, info.num_subcores
    nw = nc * ns
    rows_per_w = n_rows // nw
    chunks = rows_per_w // _CHUNK
    idx3 = gidx.reshape(nw, chunks, _CHUNK)
    mesh = plsc.VectorSubcoreMesh(core_axis_name="c", subcore_axis_name="s")

    @functools.partial(
        pl.kernel,
        mesh=mesh,
        out_type=jax.ShapeDtypeStruct((n_rows, f), jnp.float32),
        scratch_types=[
            pltpu.VMEM((chunks, _CHUNK), jnp.int32),
            pltpu.VMEM((_CHUNK, f), jnp.float32),
            pltpu.SemaphoreType.DMA,
        ],
        compiler_params=pltpu.CompilerParams(use_tc_tiling_on_sc=False),
    )
    def gather_k(idx_hbm, table_hbm, out_hbm, idx_v, rows_v, sem):
        wid = lax.axis_index("s") * nc + lax.axis_index("c")
        base = wid * rows_per_w
        pltpu.sync_copy(idx_hbm.at[wid], idx_v)

        def body(j, carry):
            pltpu.async_copy(table_hbm.at[idx_v.at[j]], rows_v, sem).wait()
            pltpu.sync_copy(rows_v, out_hbm.at[pl.ds(base + j * _CHUNK, _CHUNK)])
            return carry

        lax.fori_loop(0, chunks, body, 0)

    return gather_k(idx3, table)


def kernel(points, features, W1, b1, W2, b2):
    n, b, pdim = points.shape
    f = features.shape[-1]
    h2dim = W1.shape[0]
    h = W2.shape[0]

    pc = jnp.transpose(points, (1, 0, 2))
    pcp = jnp.pad(pc, ((0, 0), (0, 0), (0, 8 - pdim)))
    pct = jnp.transpose(pcp, (0, 2, 1))
    ft = jnp.transpose(features, (1, 0, 2))
    w1t = W1.T
    a1 = w1t[:f]
    c1 = w1t[f:] - a1
    w2t = W2.T

    gidx = pl.pallas_call(
        _topk_body,
        grid=(b,),
        in_specs=[
            pl.BlockSpec((1, n, 8), lambda i: (i, 0, 0)),
            pl.BlockSpec((1, 8, n), lambda i: (i, 0, 0)),
        ],
        out_specs=pl.BlockSpec((1, n, _K), lambda i: (i, 0, 0)),
        out_shape=jax.ShapeDtypeStruct((b, n, _K), jnp.int32),
    )(pcp, pct)

    gathered = _sc_gather(ft.reshape(b * n, f), gidx.reshape(-1), b * n * _K, f)
    gathered = gathered.reshape(b, n * _K, f)

    out = pl.pallas_call(
        _mlp_body,
        grid=(b,),
        in_specs=[
            pl.BlockSpec((1, n * _K, f), lambda i: (i, 0, 0)),
            pl.BlockSpec((1, n, f), lambda i: (i, 0, 0)),
            pl.BlockSpec((f, h2dim), lambda i: (0, 0)),
            pl.BlockSpec((f, h2dim), lambda i: (0, 0)),
            pl.BlockSpec((h2dim, h), lambda i: (0, 0)),
            pl.BlockSpec((1, h2dim), lambda i: (0, 0)),
            pl.BlockSpec((1, h), lambda i: (0, 0)),
        ],
        out_specs=pl.BlockSpec((1, n, h), lambda i: (i, 0, 0)),
        out_shape=jax.ShapeDtypeStruct((b, n, h), jnp.float32),
    )(gathered, ft, a1, c1, w2t, b1.reshape(1, -1), b2.reshape(1, -1))
    return jnp.transpose(out, (1, 0, 2))


# speedup vs baseline: 1.1494x; 1.1494x over previous
import functools

import jax
import jax.numpy as jnp
from jax import lax
from jax.experimental import pallas as pl
from jax.experimental.pallas import tpu as pltpu
from jax.experimental.pallas import tpu_sc as plsc

_K = 16
_CHUNK = 128


def _gelu(x):
    return 0.5 * x * (1.0 + lax.erf(x * 0.7071067811865476))


def _topk_body(pc_ref, pct_ref, idx_ref):
    n = pc_ref.shape[1]
    hn = n // 2
    b = pl.program_id(0)
    pcb = pc_ref[0]
    pct = pct_ref[0]

    m = jnp.dot(pcb, pct, preferred_element_type=jnp.float32)
    r_row = jnp.sum(pcb * pcb, axis=1, keepdims=True)
    r_col = jnp.sum(pct * pct, axis=0, keepdims=True)
    d = r_row - 2.0 * m + r_col + 1e-5

    inf = jnp.float32(jnp.inf)
    big = jnp.float32(2.0 * n)
    dlo, dhi = d[:, :hn], d[:, hn:]
    colsf = lax.broadcasted_iota(jnp.int32, (n, hn), 1).astype(jnp.float32)
    side = dlo <= dhi
    dm = jnp.minimum(dlo, dhi)
    ds = jnp.maximum(dlo, dhi)
    im = jnp.where(side, colsf, colsf + hn)
    pairsum = 2.0 * colsf + hn
    ams = []
    for t in range(_K + 1):
        minval = jnp.min(dm, axis=1, keepdims=True)
        tt = jnp.where(dm == minval, im, big)
        am = jnp.min(tt, axis=1, keepdims=True)
        if t > 0:
            ams.append(am)
        selp = tt == am
        dm = jnp.where(selp, ds, dm)
        im = jnp.where(selp, pairsum - im, im)
        ds = jnp.where(selp, inf, ds)
    idxf = jnp.concatenate(ams, axis=1)
    idx_ref[0] = idxf.astype(jnp.int32) + b * n


def _mlp_body(g_ref, ft_ref, a1_ref, c1_ref, w2t_ref, b1_ref, b2_ref,
              out_ref):
    n = ft_ref.shape[1]
    knn = g_ref[0]
    ftb = ft_ref[0]
    c = jnp.dot(ftb, c1_ref[...], preferred_element_type=jnp.float32) + b1_ref[...]
    h1 = jnp.dot(knn, a1_ref[...], preferred_element_type=jnp.float32)
    h1 = _gelu(h1.reshape(n, _K, -1) + c[:, None, :])
    h2 = jnp.dot(h1.reshape(n * _K, -1), w2t_ref[...],
                 preferred_element_type=jnp.float32) + b2_ref[...]
    h2 = _gelu(h2)
    out_ref[0] = jnp.mean(h2.reshape(n, _K, -1), axis=1)


def _sc_gather(table, gidx, n_rows, f):
    info = plsc.get_sparse_core_info()
    nc, ns = info.num_cores, info.num_subcores
    nw = nc * ns
    rows_per_w = n_rows // nw
    chunks = rows_per_w // _CHUNK
    idx3 = gidx.reshape(nw, chunks, _CHUNK)
    mesh = plsc.VectorSubcoreMesh(core_axis_name="c", subcore_axis_name="s")

    @functools.partial(
        pl.kernel,
        mesh=mesh,
        out_type=jax.ShapeDtypeStruct((n_rows, f), jnp.float32),
        scratch_types=[
            pltpu.VMEM((chunks, _CHUNK), jnp.int32),
            pltpu.VMEM((_CHUNK, f), jnp.float32),
            pltpu.SemaphoreType.DMA,
        ],
        compiler_params=pltpu.CompilerParams(use_tc_tiling_on_sc=False),
    )
    def gather_k(idx_hbm, table_hbm, out_hbm, idx_v, rows_v, sem):
        wid = lax.axis_index("s") * nc + lax.axis_index("c")
        base = wid * rows_per_w
        pltpu.sync_copy(idx_hbm.at[wid], idx_v)

        def body(j, carry):
            pltpu.async_copy(table_hbm.at[idx_v.at[j]], rows_v, sem).wait()
            pltpu.sync_copy(rows_v, out_hbm.at[pl.ds(base + j * _CHUNK, _CHUNK)])
            return carry

        lax.fori_loop(0, chunks, body, 0)

    return gather_k(idx3, table)


def kernel(points, features, W1, b1, W2, b2):
    n, b, pdim = points.shape
    f = features.shape[-1]
    h2dim = W1.shape[0]
    h = W2.shape[0]

    pc = jnp.transpose(points, (1, 0, 2))
    pcp = jnp.pad(pc, ((0, 0), (0, 0), (0, 8 - pdim)))
    pct = jnp.transpose(pcp, (0, 2, 1))
    ft = jnp.transpose(features, (1, 0, 2))
    w1t = W1.T
    a1 = w1t[:f]
    c1 = w1t[f:] - a1
    w2t = W2.T

    gidx = pl.pallas_call(
        _topk_body,
        grid=(b,),
        in_specs=[
            pl.BlockSpec((1, n, 8), lambda i: (i, 0, 0)),
            pl.BlockSpec((1, 8, n), lambda i: (i, 0, 0)),
        ],
        out_specs=pl.BlockSpec((1, n, _K), lambda i: (i, 0, 0)),
        out_shape=jax.ShapeDtypeStruct((b, n, _K), jnp.int32),
    )(pcp, pct)

    gathered = _sc_gather(ft.reshape(b * n, f), gidx.reshape(-1), b * n * _K, f)
    gathered = gathered.reshape(b, n * _K, f)

    out = pl.pallas_call(
        _mlp_body,
        grid=(b,),
        in_specs=[
            pl.BlockSpec((1, n * _K, f), lambda i: (i, 0, 0)),
            pl.BlockSpec((1, n, f), lambda i: (i, 0, 0)),
            pl.BlockSpec((f, h2dim), lambda i: (0, 0)),
            pl.BlockSpec((f, h2dim), lambda i: (0, 0)),
            pl.BlockSpec((h2dim, h), lambda i: (0, 0)),
            pl.BlockSpec((1, h2dim), lambda i: (0, 0)),
            pl.BlockSpec((1, h), lambda i: (0, 0)),
        ],
        out_specs=pl.BlockSpec((1, n, h), lambda i: (i, 0, 0)),
        out_shape=jax.ShapeDtypeStruct((b, n, h), jnp.float32),
    )(gathered, ft, a1, c1, w2t, b1.reshape(1, -1), b2.reshape(1, -1))
    return jnp.transpose(out, (1, 0, 2))


# speedup vs baseline: 1.2744x; 1.1088x over previous
import functools

import jax
import jax.numpy as jnp
from jax import lax
from jax.experimental import pallas as pl
from jax.experimental.pallas import tpu as pltpu
from jax.experimental.pallas import tpu_sc as plsc

_K = 16
_CHUNK = 128
_NBUF = 4


def _gelu(x):
    return 0.5 * x * (1.0 + lax.erf(x * 0.7071067811865476))


def _topk_body(pc_ref, pct_ref, idx_ref):
    n = pc_ref.shape[1]
    hn = n // 2
    b = pl.program_id(0)
    pcb = pc_ref[0]
    pct = pct_ref[0]

    m = jnp.dot(pcb, pct, preferred_element_type=jnp.float32)
    r_row = jnp.sum(pcb * pcb, axis=1, keepdims=True)
    r_col = jnp.sum(pct * pct, axis=0, keepdims=True)
    d = r_row - 2.0 * m + r_col + 1e-5

    inf = jnp.float32(jnp.inf)
    big = jnp.float32(2.0 * n)
    dlo, dhi = d[:, :hn], d[:, hn:]
    colsf = lax.broadcasted_iota(jnp.int32, (n, hn), 1).astype(jnp.float32)
    side = dlo <= dhi
    dm = jnp.minimum(dlo, dhi)
    ds = jnp.maximum(dlo, dhi)
    im = jnp.where(side, colsf, colsf + hn)
    pairsum = 2.0 * colsf + hn
    ams = []
    for t in range(_K + 1):
        minval = jnp.min(dm, axis=1, keepdims=True)
        tt = jnp.where(dm == minval, im, big)
        am = jnp.min(tt, axis=1, keepdims=True)
        if t > 0:
            ams.append(am)
        selp = tt == am
        dm = jnp.where(selp, ds, dm)
        im = jnp.where(selp, pairsum - im, im)
        ds = jnp.where(selp, inf, ds)
    idxf = jnp.concatenate(ams, axis=1)
    idx_ref[0] = idxf.astype(jnp.int32) + b * n


def _mlp_body(g_ref, ft_ref, a1_ref, c1_ref, w2t_ref, b1_ref, b2_ref,
              out_ref):
    n = ft_ref.shape[1]
    knn = g_ref[0]
    ftb = ft_ref[0]
    c = jnp.dot(ftb, c1_ref[...], preferred_element_type=jnp.float32) + b1_ref[...]
    h1 = jnp.dot(knn, a1_ref[...], preferred_element_type=jnp.float32)
    h1 = _gelu(h1.reshape(n, _K, -1) + c[:, None, :])
    h2 = jnp.dot(h1.reshape(n * _K, -1), w2t_ref[...],
                 preferred_element_type=jnp.float32) + b2_ref[...]
    h2 = _gelu(h2)
    out_ref[0] = jnp.mean(h2.reshape(n, _K, -1), axis=1)


def _sc_gather(table, gidx, n_rows, f):
    info = plsc.get_sparse_core_info()
    nc, ns = info.num_cores, info.num_subcores
    nw = nc * ns
    rows_per_w = n_rows // nw
    chunks = rows_per_w // _CHUNK
    idx3 = gidx.reshape(nw, chunks, _CHUNK)
    mesh = plsc.VectorSubcoreMesh(core_axis_name="c", subcore_axis_name="s")

    @functools.partial(
        pl.kernel,
        mesh=mesh,
        out_type=jax.ShapeDtypeStruct((n_rows, f), jnp.float32),
        scratch_types=[
            pltpu.VMEM((chunks, _CHUNK), jnp.int32),
            pltpu.VMEM((_NBUF, _CHUNK, f), jnp.float32),
            pltpu.SemaphoreType.DMA((_NBUF,)),
            pltpu.SemaphoreType.DMA((_NBUF,)),
        ],
        compiler_params=pltpu.CompilerParams(use_tc_tiling_on_sc=False),
    )
    def gather_k(idx_hbm, table_hbm, out_hbm, idx_v, rows_v, gsem, wsem):
        wid = lax.axis_index("s") * nc + lax.axis_index("c")
        base = wid * rows_per_w
        pltpu.sync_copy(idx_hbm.at[wid], idx_v)

        def body(i, carry):
            gs = []
            for t in range(_NBUF):
                j = i * _NBUF + t
                gs.append(pltpu.async_copy(
                    table_hbm.at[idx_v.at[j]], rows_v.at[t], gsem.at[t]))
            ws = []
            for t in range(_NBUF):
                j = i * _NBUF + t
                gs[t].wait()
                ws.append(pltpu.async_copy(
                    rows_v.at[t], out_hbm.at[pl.ds(base + j * _CHUNK, _CHUNK)],
                    wsem.at[t]))
            for t in range(_NBUF):
                ws[t].wait()
            return carry

        lax.fori_loop(0, chunks // _NBUF, body, 0)

    return gather_k(idx3, table)


def kernel(points, features, W1, b1, W2, b2):
    n, b, pdim = points.shape
    f = features.shape[-1]
    h2dim = W1.shape[0]
    h = W2.shape[0]

    pc = jnp.transpose(points, (1, 0, 2))
    pcp = jnp.pad(pc, ((0, 0), (0, 0), (0, 8 - pdim)))
    pct = jnp.transpose(pcp, (0, 2, 1))
    ft = jnp.transpose(features, (1, 0, 2))
    w1t = W1.T
    a1 = w1t[:f]
    c1 = w1t[f:] - a1
    w2t = W2.T

    gidx = pl.pallas_call(
        _topk_body,
        grid=(b,),
        in_specs=[
            pl.BlockSpec((1, n, 8), lambda i: (i, 0, 0)),
            pl.BlockSpec((1, 8, n), lambda i: (i, 0, 0)),
        ],
        out_specs=pl.BlockSpec((1, n, _K), lambda i: (i, 0, 0)),
        out_shape=jax.ShapeDtypeStruct((b, n, _K), jnp.int32),
    )(pcp, pct)

    gathered = _sc_gather(ft.reshape(b * n, f), gidx.reshape(-1), b * n * _K, f)
    gathered = gathered.reshape(b, n * _K, f)

    out = pl.pallas_call(
        _mlp_body,
        grid=(b,),
        in_specs=[
            pl.BlockSpec((1, n * _K, f), lambda i: (i, 0, 0)),
            pl.BlockSpec((1, n, f), lambda i: (i, 0, 0)),
            pl.BlockSpec((f, h2dim), lambda i: (0, 0)),
            pl.BlockSpec((f, h2dim), lambda i: (0, 0)),
            pl.BlockSpec((h2dim, h), lambda i: (0, 0)),
            pl.BlockSpec((1, h2dim), lambda i: (0, 0)),
            pl.BlockSpec((1, h), lambda i: (0, 0)),
        ],
        out_specs=pl.BlockSpec((1, n, h), lambda i: (i, 0, 0)),
        out_shape=jax.ShapeDtypeStruct((b, n, h), jnp.float32),
    )(gathered, ft, a1, c1, w2t, b1.reshape(1, -1), b2.reshape(1, -1))
    return jnp.transpose(out, (1, 0, 2))


# speedup vs baseline: 1.3261x; 1.0405x over previous
import functools

import jax
import jax.numpy as jnp
from jax import lax
from jax.experimental import pallas as pl
from jax.experimental.pallas import tpu as pltpu
from jax.experimental.pallas import tpu_sc as plsc

_K = 16
_CHUNK = 128
_NBUF = 4


def _gelu(x):
    return 0.5 * x * (1.0 + lax.erf(x * 0.7071067811865476))


def _topk_body(pc_ref, pct_ref, idx_ref, *, base):
    n = pc_ref.shape[1]
    hn = n // 2
    b = pl.program_id(0) + base
    pcb = pc_ref[0]
    pct = pct_ref[0]

    m = jnp.dot(pcb, pct, preferred_element_type=jnp.float32)
    r_row = jnp.sum(pcb * pcb, axis=1, keepdims=True)
    r_col = jnp.sum(pct * pct, axis=0, keepdims=True)
    d = r_row - 2.0 * m + r_col + 1e-5

    inf = jnp.float32(jnp.inf)
    big = jnp.float32(2.0 * n)
    dlo, dhi = d[:, :hn], d[:, hn:]
    colsf = lax.broadcasted_iota(jnp.int32, (n, hn), 1).astype(jnp.float32)
    side = dlo <= dhi
    dm = jnp.minimum(dlo, dhi)
    ds = jnp.maximum(dlo, dhi)
    im = jnp.where(side, colsf, colsf + hn)
    pairsum = 2.0 * colsf + hn
    ams = []
    for t in range(_K + 1):
        minval = jnp.min(dm, axis=1, keepdims=True)
        tt = jnp.where(dm == minval, im, big)
        am = jnp.min(tt, axis=1, keepdims=True)
        if t > 0:
            ams.append(am)
        selp = tt == am
        dm = jnp.where(selp, ds, dm)
        im = jnp.where(selp, pairsum - im, im)
        ds = jnp.where(selp, inf, ds)
    idxf = jnp.concatenate(ams, axis=1)
    idx_ref[0] = idxf.astype(jnp.int32) + b * n


def _mlp_body(g_ref, ft_ref, a1_ref, c1_ref, w2t_ref, b1_ref, b2_ref,
              out_ref):
    n = ft_ref.shape[1]
    knn = g_ref[0]
    ftb = ft_ref[0]
    c = jnp.dot(ftb, c1_ref[...], preferred_element_type=jnp.float32) + b1_ref[...]
    h1 = jnp.dot(knn, a1_ref[...], preferred_element_type=jnp.float32)
    h1 = _gelu(h1.reshape(n, _K, -1) + c[:, None, :])
    h2 = jnp.dot(h1.reshape(n * _K, -1), w2t_ref[...],
                 preferred_element_type=jnp.float32) + b2_ref[...]
    h2 = _gelu(h2)
    out_ref[0] = jnp.mean(h2.reshape(n, _K, -1), axis=1)


def _sc_gather(table, gidx, n_rows, f):
    info = plsc.get_sparse_core_info()
    nc, ns = info.num_cores, info.num_subcores
    nw = nc * ns
    rows_per_w = n_rows // nw
    chunks = rows_per_w // _CHUNK
    idx3 = gidx.reshape(nw, chunks, _CHUNK)
    mesh = plsc.VectorSubcoreMesh(core_axis_name="c", subcore_axis_name="s")

    @functools.partial(
        pl.kernel,
        mesh=mesh,
        out_type=jax.ShapeDtypeStruct((n_rows, f), jnp.float32),
        scratch_types=[
            pltpu.VMEM((chunks, _CHUNK), jnp.int32),
            pltpu.VMEM((_NBUF, _CHUNK, f), jnp.float32),
            pltpu.SemaphoreType.DMA((_NBUF,)),
            pltpu.SemaphoreType.DMA((_NBUF,)),
        ],
        compiler_params=pltpu.CompilerParams(use_tc_tiling_on_sc=False),
    )
    def gather_k(idx_hbm, table_hbm, out_hbm, idx_v, rows_v, gsem, wsem):
        wid = lax.axis_index("s") * nc + lax.axis_index("c")
        base = wid * rows_per_w
        pltpu.sync_copy(idx_hbm.at[wid], idx_v)

        def body(i, carry):
            gs = []
            for t in range(_NBUF):
                j = i * _NBUF + t
                gs.append(pltpu.async_copy(
                    table_hbm.at[idx_v.at[j]], rows_v.at[t], gsem.at[t]))
            ws = []
            for t in range(_NBUF):
                j = i * _NBUF + t
                gs[t].wait()
                ws.append(pltpu.async_copy(
                    rows_v.at[t], out_hbm.at[pl.ds(base + j * _CHUNK, _CHUNK)],
                    wsem.at[t]))
            for t in range(_NBUF):
                ws[t].wait()
            return carry

        lax.fori_loop(0, chunks // _NBUF, body, 0)

    return gather_k(idx3, table)


def kernel(points, features, W1, b1, W2, b2):
    n, b, pdim = points.shape
    f = features.shape[-1]
    h2dim = W1.shape[0]
    h = W2.shape[0]

    pc = jnp.transpose(points, (1, 0, 2))
    pcp = jnp.pad(pc, ((0, 0), (0, 0), (0, 8 - pdim)))
    pct = jnp.transpose(pcp, (0, 2, 1))
    ft = jnp.transpose(features, (1, 0, 2))
    w1t = W1.T
    a1 = w1t[:f]
    c1 = w1t[f:] - a1
    w2t = W2.T

    ft2d = ft.reshape(b * n, f)
    b1r, b2r = b1.reshape(1, -1), b2.reshape(1, -1)
    ns = 4 if b % 4 == 0 else 1
    bs = b // ns

    outs = []
    for s in range(ns):
        gidx = pl.pallas_call(
            functools.partial(_topk_body, base=s * bs),
            grid=(bs,),
            in_specs=[
                pl.BlockSpec((1, n, 8), lambda i, s=s: (i + s * bs, 0, 0)),
                pl.BlockSpec((1, 8, n), lambda i, s=s: (i + s * bs, 0, 0)),
            ],
            out_specs=pl.BlockSpec((1, n, _K), lambda i: (i, 0, 0)),
            out_shape=jax.ShapeDtypeStruct((bs, n, _K), jnp.int32),
        )(pcp, pct)

        gathered = _sc_gather(ft2d, gidx.reshape(-1), bs * n * _K, f)
        gathered = gathered.reshape(bs, n * _K, f)

        outs.append(pl.pallas_call(
            _mlp_body,
            grid=(bs,),
            in_specs=[
                pl.BlockSpec((1, n * _K, f), lambda i: (i, 0, 0)),
                pl.BlockSpec((1, n, f), lambda i, s=s: (i + s * bs, 0, 0)),
                pl.BlockSpec((f, h2dim), lambda i: (0, 0)),
                pl.BlockSpec((f, h2dim), lambda i: (0, 0)),
                pl.BlockSpec((h2dim, h), lambda i: (0, 0)),
                pl.BlockSpec((1, h2dim), lambda i: (0, 0)),
                pl.BlockSpec((1, h), lambda i: (0, 0)),
            ],
            out_specs=pl.BlockSpec((1, n, h), lambda i: (i, 0, 0)),
            out_shape=jax.ShapeDtypeStruct((bs, n, h), jnp.float32),
        )(gathered, ft, a1, c1, w2t, b1r, b2r))
    out = outs[0] if ns == 1 else jnp.concatenate(outs, axis=0)
    return jnp.transpose(out, (1, 0, 2))


# speedup vs baseline: 1.7696x; 1.3345x over previous
import functools

import jax
import jax.numpy as jnp
from jax import lax
from jax.experimental import pallas as pl
from jax.experimental.pallas import tpu as pltpu
from jax.experimental.pallas import tpu_sc as plsc

_K = 16
_CHUNK = 128
_NBUF = 4


def _gelu(x):
    return 0.5 * x * (1.0 + lax.erf(x * 0.7071067811865476))


def _topk_body(pc_ref, pct_ref, idx_ref, *, base):
    n = pc_ref.shape[1]
    hn = n // 2
    b = pl.program_id(0) + base
    pcb = pc_ref[0]
    pct = pct_ref[0]

    m = jnp.dot(pcb, pct, preferred_element_type=jnp.float32)
    r_row = jnp.sum(pcb * pcb, axis=1, keepdims=True)
    r_col = jnp.sum(pct * pct, axis=0, keepdims=True)
    d = r_row - 2.0 * m + r_col + 1e-5

    inf = jnp.float32(jnp.inf)
    big = jnp.float32(2.0 * n)
    dlo, dhi = d[:, :hn], d[:, hn:]
    colsf = lax.broadcasted_iota(jnp.int32, (n, hn), 1).astype(jnp.float32)
    side = dlo <= dhi
    dm = jnp.minimum(dlo, dhi)
    ds = jnp.maximum(dlo, dhi)
    im = jnp.where(side, colsf, colsf + hn)
    pairsum = 2.0 * colsf + hn
    ams = []
    for t in range(_K + 1):
        minval = jnp.min(dm, axis=1, keepdims=True)
        tt = jnp.where(dm == minval, im, big)
        am = jnp.min(tt, axis=1, keepdims=True)
        if t > 0:
            ams.append(am)
        selp = tt == am
        dm = jnp.where(selp, ds, dm)
        im = jnp.where(selp, pairsum - im, im)
        ds = jnp.where(selp, inf, ds)
    idxf = jnp.concatenate(ams, axis=1)
    idx_ref[0] = idxf.astype(jnp.int32) + b * n


def _mlp_body(g_ref, ft_ref, a1_ref, c1_ref, w2t_ref, b1_ref, b2_ref,
              out_ref):
    n = ft_ref.shape[1]
    f = ft_ref.shape[2]
    h = out_ref.shape[2]
    per = 128 // f
    gw = n * _K * f // 128
    gpp = gw // n
    packed = g_ref[0]
    knn4 = jnp.concatenate([packed[:, q * f:(q + 1) * f] for q in range(per)],
                           axis=0)
    ftb = ft_ref[0]
    c = jnp.dot(ftb, c1_ref[...], preferred_element_type=jnp.float32) + b1_ref[...]
    cb = jnp.broadcast_to(c[:, None, :], (n, gpp, c.shape[-1]))
    cb = cb.reshape(n * gpp, -1)
    cb = jnp.concatenate([cb] * per, axis=0)
    h1 = jnp.dot(knn4, a1_ref[...], preferred_element_type=jnp.float32)
    h1 = _gelu(h1 + cb)
    h2 = jnp.dot(h1, w2t_ref[...],
                 preferred_element_type=jnp.float32) + b2_ref[...]
    h2 = _gelu(h2)
    s = jnp.sum(h2.reshape(per, gw, h), axis=0)
    s = jnp.sum(s.reshape(n, gpp, h), axis=1)
    out_ref[0] = s * (1.0 / _K)


def _sc_gather(table, gidx, n_rows, f):
    info = plsc.get_sparse_core_info()
    nc, ns = info.num_cores, info.num_subcores
    nw = nc * ns
    rows_per_w = n_rows // nw
    chunks = rows_per_w // _CHUNK
    idx3 = gidx.reshape(nw, chunks, _CHUNK)
    mesh = plsc.VectorSubcoreMesh(core_axis_name="c", subcore_axis_name="s")

    @functools.partial(
        pl.kernel,
        mesh=mesh,
        out_type=jax.ShapeDtypeStruct((n_rows, f), jnp.float32),
        scratch_types=[
            pltpu.VMEM((chunks, _CHUNK), jnp.int32),
            pltpu.VMEM((_NBUF, _CHUNK, f), jnp.float32),
            pltpu.SemaphoreType.DMA((_NBUF,)),
            pltpu.SemaphoreType.DMA((_NBUF,)),
        ],
        compiler_params=pltpu.CompilerParams(use_tc_tiling_on_sc=False),
    )
    def gather_k(idx_hbm, table_hbm, out_hbm, idx_v, rows_v, gsem, wsem):
        wid = lax.axis_index("s") * nc + lax.axis_index("c")
        base = wid * rows_per_w
        pltpu.sync_copy(idx_hbm.at[wid], idx_v)

        def body(i, carry):
            gs = []
            for t in range(_NBUF):
                j = i * _NBUF + t
                gs.append(pltpu.async_copy(
                    table_hbm.at[idx_v.at[j]], rows_v.at[t], gsem.at[t]))
            ws = []
            for t in range(_NBUF):
                j = i * _NBUF + t
                gs[t].wait()
                ws.append(pltpu.async_copy(
                    rows_v.at[t], out_hbm.at[pl.ds(base + j * _CHUNK, _CHUNK)],
                    wsem.at[t]))
            for t in range(_NBUF):
                ws[t].wait()
            return carry

        lax.fori_loop(0, chunks // _NBUF, body, 0)

    return gather_k(idx3, table)


def kernel(points, features, W1, b1, W2, b2):
    n, b, pdim = points.shape
    f = features.shape[-1]
    h2dim = W1.shape[0]
    h = W2.shape[0]

    pc = jnp.transpose(points, (1, 0, 2))
    pcp = jnp.pad(pc, ((0, 0), (0, 0), (0, 8 - pdim)))
    pct = jnp.transpose(pcp, (0, 2, 1))
    ft = jnp.transpose(features, (1, 0, 2))
    w1t = W1.T
    a1 = w1t[:f]
    c1 = w1t[f:] - a1
    w2t = W2.T

    ft2d = ft.reshape(b * n, f)
    b1r, b2r = b1.reshape(1, -1), b2.reshape(1, -1)
    ns = 4 if b % 4 == 0 else 1
    bs = b // ns

    outs = []
    for s in range(ns):
        gidx = pl.pallas_call(
            functools.partial(_topk_body, base=s * bs),
            grid=(bs,),
            in_specs=[
                pl.BlockSpec((1, n, 8), lambda i, s=s: (i + s * bs, 0, 0)),
                pl.BlockSpec((1, 8, n), lambda i, s=s: (i + s * bs, 0, 0)),
            ],
            out_specs=pl.BlockSpec((1, n, _K), lambda i: (i, 0, 0)),
            out_shape=jax.ShapeDtypeStruct((bs, n, _K), jnp.int32),
        )(pcp, pct)

        gathered = _sc_gather(ft2d, gidx.reshape(-1), bs * n * _K, f)
        gw = n * _K * f // 128
        gathered = gathered.reshape(bs, gw, 128)

        outs.append(pl.pallas_call(
            _mlp_body,
            grid=(bs,),
            in_specs=[
                pl.BlockSpec((1, gw, 128), lambda i: (i, 0, 0)),
                pl.BlockSpec((1, n, f), lambda i, s=s: (i + s * bs, 0, 0)),
                pl.BlockSpec((f, h2dim), lambda i: (0, 0)),
                pl.BlockSpec((f, h2dim), lambda i: (0, 0)),
                pl.BlockSpec((h2dim, h), lambda i: (0, 0)),
                pl.BlockSpec((1, h2dim), lambda i: (0, 0)),
                pl.BlockSpec((1, h), lambda i: (0, 0)),
            ],
            out_specs=pl.BlockSpec((1, n, h), lambda i: (i, 0, 0)),
            out_shape=jax.ShapeDtypeStruct((bs, n, h), jnp.float32),
        )(gathered, ft, a1, c1, w2t, b1r, b2r))
    out = outs[0] if ns == 1 else jnp.concatenate(outs, axis=0)
    return jnp.transpose(out, (1, 0, 2))
